# Initial kernel scaffold; baseline (speedup 1.0000x reference)
import jax
import jax.numpy as jnp
from jax.experimental import pallas as pl


def kernel(x_src, x_dst, W_l, b_l, W_r, edge_index):
    raise NotImplementedError("write your pallas kernel here")


# speedup vs baseline: 3.1418x; 3.1418x over previous
import functools

import jax
import jax.numpy as jnp
from jax import lax
from jax.experimental import pallas as pl
from jax.experimental.pallas import tpu as pltpu
from jax.experimental.pallas import tpu_sc as plsc

N_NODES = 10000
N_EDGES = 320000
D = 128

NC = 2
NS = 16
E_PER_T = N_EDGES // NS
CHUNK = 80
N_CHUNKS = E_PER_T // CHUNK
N_PAD = 10240
HALF = N_PAD // NC
ROWS_PER_TILE = HALF // NS
ZROWS = 40
IGN = 2**31 - 1


def _sc_aggregate(x_src, row_idx, col_idx, ones):
    mesh = plsc.VectorSubcoreMesh(core_axis_name="c", subcore_axis_name="s")

    @functools.partial(
        pl.kernel,
        out_type=(
            jax.ShapeDtypeStruct((NC, HALF, D), jnp.float32),
            jax.ShapeDtypeStruct((NC, HALF, D), jnp.float32),
        ),
        mesh=mesh,
        scratch_types=[
            pltpu.VMEM((CHUNK,), jnp.int32),
            pltpu.VMEM((CHUNK,), jnp.int32),
            pltpu.VMEM((CHUNK,), jnp.int32),
            pltpu.VMEM((CHUNK,), jnp.int32),
            pltpu.VMEM((CHUNK, D), jnp.float32),
            pltpu.VMEM((CHUNK, D), jnp.float32),
            pltpu.VMEM((ZROWS, D), jnp.float32),
            pltpu.VMEM_SHARED((HALF, D), jnp.float32),
            pltpu.VMEM_SHARED((HALF, D), jnp.float32),
            pltpu.SemaphoreType.DMA,
        ],
    )
    def sc_kernel(x_hbm, row_hbm, col_hbm, ones_hbm, acc_out, cnt_out,
                  row_v, col_v, selr_v, selc_v, rows_v, ones_v, zbuf_v,
                  acc_s, cnt_s, sem):
        cid = lax.axis_index("c")
        sid = lax.axis_index("s")
        zeros16 = jnp.zeros((16,), jnp.float32)
        row_base = sid * ROWS_PER_TILE
        lo = cid * HALF

        pltpu.sync_copy(ones_hbm, ones_v)

        @pl.loop(0, ZROWS)
        def _(r):
            for j in range(D // 16):
                zbuf_v[r, pl.ds(j * 16, 16)] = zeros16

        @pl.loop(0, ROWS_PER_TILE // ZROWS)
        def _(i):
            pltpu.sync_copy(zbuf_v, acc_s.at[pl.ds(row_base + i * ZROWS, ZROWS)])
            pltpu.sync_copy(zbuf_v, cnt_s.at[pl.ds(row_base + i * ZROWS, ZROWS)])

        plsc.subcore_barrier()

        e_base = sid * E_PER_T

        @pl.loop(0, N_CHUNKS)
        def _(g):
            off = e_base + g * CHUNK
            pltpu.sync_copy(row_hbm.at[pl.ds(off, CHUNK)], row_v)
            pltpu.sync_copy(col_hbm.at[pl.ds(off, CHUNK)], col_v)
            for j in range(CHUNK // 16):
                lanes = pl.ds(j * 16, 16)
                r16 = row_v[lanes]
                c16 = col_v[lanes]
                owned = (c16 >= lo) & (c16 < lo + HALF)
                selr_v[lanes] = jnp.where(owned, r16, IGN)
                selc_v[lanes] = jnp.where(owned, c16 - lo, IGN)
            pltpu.async_copy(x_hbm.at[plsc.Indices(selr_v, ignored_value=IGN)],
                             rows_v, sem).wait()
            pltpu.sync_copy(rows_v,
                            acc_s.at[plsc.Indices(selc_v, ignored_value=IGN)],
                            add=True)
            pltpu.sync_copy(ones_v,
                            cnt_s.at[plsc.Indices(selc_v, ignored_value=IGN)],
                            add=True)

        plsc.subcore_barrier()

        rows = pl.ds(row_base, ROWS_PER_TILE)
        pltpu.sync_copy(acc_s.at[rows], acc_out.at[cid, rows])
        pltpu.sync_copy(cnt_s.at[rows], cnt_out.at[cid, rows])

    return sc_kernel(x_src, row_idx, col_idx, ones)


BN = 2000


def _tc_combine_kernel(acc_ref, cnt_ref, xd_ref, wl_ref, bl_ref, wr_ref, o_ref):
    cnt = cnt_ref[:, 0:1]
    aggr = acc_ref[...] / jnp.maximum(cnt, 1.0)
    dn = (((1,), (1,)), ((), ()))
    out = lax.dot_general(aggr, wl_ref[...], dn, preferred_element_type=jnp.float32)
    out += lax.dot_general(xd_ref[...], wr_ref[...], dn, preferred_element_type=jnp.float32)
    o_ref[...] = out + bl_ref[...]


def _tc_combine(acc, cnt, x_dst, W_l, b_l, W_r):
    return pl.pallas_call(
        _tc_combine_kernel,
        grid=(N_NODES // BN,),
        in_specs=[
            pl.BlockSpec((BN, D), lambda i: (i, 0)),
            pl.BlockSpec((BN, D), lambda i: (i, 0)),
            pl.BlockSpec((BN, D), lambda i: (i, 0)),
            pl.BlockSpec((D, D), lambda i: (0, 0)),
            pl.BlockSpec((1, D), lambda i: (0, 0)),
            pl.BlockSpec((D, D), lambda i: (0, 0)),
        ],
        out_specs=pl.BlockSpec((BN, D), lambda i: (i, 0)),
        out_shape=jax.ShapeDtypeStruct((N_NODES, D), jnp.float32),
    )(acc, cnt, x_dst, W_l, b_l, W_r)


def kernel(x_src, x_dst, W_l, b_l, W_r, edge_index):
    row_idx = edge_index[0].astype(jnp.int32)
    col_idx = edge_index[1].astype(jnp.int32)
    ones = jnp.ones((CHUNK, D), jnp.float32)
    acc, cnt = _sc_aggregate(x_src, row_idx, col_idx, ones)
    acc = acc.reshape(N_PAD, D)
    cnt = cnt.reshape(N_PAD, D)
    return _tc_combine(acc, cnt, x_dst, W_l, b_l.reshape(1, D), W_r)


# speedup vs baseline: 8.9062x; 2.8347x over previous
import functools

import jax
import jax.numpy as jnp
from jax import lax
from jax.experimental import pallas as pl
from jax.experimental.pallas import tpu as pltpu
from jax.experimental.pallas import tpu_sc as plsc

N_NODES = 10000
N_EDGES = 320000
D = 128

NC = 2
NS = 16
E_PER_T = N_EDGES // NS
CHUNK = 80
N_CHUNKS = E_PER_T // CHUNK
RB = 4
N_SUPER = (N_CHUNKS - 2) // RB
LAG = 2
N_PAD = 10240
HALF = N_PAD // NC
ROWS_PER_TILE = HALF // NS
ZROWS = 40
IGN = 2**31 - 1


def _sc_aggregate(x_src, row_idx, col_idx):
    mesh = plsc.VectorSubcoreMesh(core_axis_name="c", subcore_axis_name="s")

    @functools.partial(
        pl.kernel,
        out_type=(
            jax.ShapeDtypeStruct((NC, HALF, D), jnp.float32),
            jax.ShapeDtypeStruct((NC, HALF, D), jnp.float32),
        ),
        mesh=mesh,
        scratch_types=[
            [pltpu.VMEM((CHUNK,), jnp.int32)] * RB,
            [pltpu.VMEM((CHUNK,), jnp.int32)] * RB,
            [pltpu.VMEM((CHUNK,), jnp.int32)] * RB,
            [pltpu.VMEM((CHUNK,), jnp.int32)] * RB,
            [pltpu.VMEM((CHUNK, D), jnp.float32)] * RB,
            pltpu.VMEM((ZROWS, D), jnp.float32),
            pltpu.VMEM_SHARED((HALF, D), jnp.float32),
            pltpu.VMEM_SHARED((HALF, D), jnp.float32),
            [pltpu.SemaphoreType.DMA] * RB,
            [pltpu.SemaphoreType.DMA] * RB,
            [pltpu.SemaphoreType.DMA] * RB,
            [pltpu.SemaphoreType.DMA] * RB,
        ],
    )
    def sc_kernel(x_hbm, row_hbm, col_hbm, acc_out, cnt_out,
                  rowi, coli, selr, selc, rows, zbuf_v,
                  acc_s, cnt_s, isem, gsem, asem, csem):
        cid = lax.axis_index("c")
        sid = lax.axis_index("s")
        zeros16 = jnp.zeros((16,), jnp.float32)
        ones16 = jnp.ones((16,), jnp.float32)
        row_base = sid * ROWS_PER_TILE
        lo = cid * HALF
        e_base = sid * E_PER_T

        def start_idx(g, k, col_only=False):
            sl = pl.ds(e_base + g * CHUNK, CHUNK)
            pltpu.async_copy(col_hbm.at[sl], coli[k], isem[k])
            if not col_only:
                pltpu.async_copy(row_hbm.at[sl], rowi[k], isem[k])

        def wait_idx(k, col_only=False):
            pltpu.make_async_copy(col_hbm.at[pl.ds(0, CHUNK)], coli[k],
                                  isem[k]).wait()
            if not col_only:
                pltpu.make_async_copy(row_hbm.at[pl.ds(0, CHUNK)], rowi[k],
                                      isem[k]).wait()

        def compute_sel(k, col_only=False):
            for j in range(CHUNK // 16):
                lanes = pl.ds(j * 16, 16)
                c16 = coli[k][lanes]
                owned = (c16 >= lo) & (c16 < lo + HALF)
                selc[k][lanes] = jnp.where(owned, c16 - lo, IGN)
                if not col_only:
                    selr[k][lanes] = jnp.where(owned, rowi[k][lanes], IGN)

        def gather_desc(k):
            return (x_hbm.at[plsc.Indices(selr[k], ignored_value=IGN)],
                    rows[k], gsem[k])

        def acc_desc(k):
            return (rows[k], acc_s.at[plsc.Indices(selc[k], ignored_value=IGN)],
                    asem[k])

        def cnt_desc(k):
            return (rows[0], cnt_s.at[plsc.Indices(selc[k], ignored_value=IGN)],
                    csem[k])

        @pl.loop(0, ZROWS)
        def _(r):
            for j in range(D // 16):
                zbuf_v[r, pl.ds(j * 16, 16)] = zeros16

        @pl.loop(0, ROWS_PER_TILE // ZROWS)
        def _(i):
            pltpu.sync_copy(zbuf_v, acc_s.at[pl.ds(row_base + i * ZROWS, ZROWS)])
            pltpu.sync_copy(zbuf_v, cnt_s.at[pl.ds(row_base + i * ZROWS, ZROWS)])

        plsc.subcore_barrier()

        for g in range(LAG):
            start_idx(g, g)

        @pl.loop(0, N_SUPER)
        def _(G):
            for k in range(RB):
                g = G * RB + k

                @pl.when(G >= 1)
                def _():
                    pltpu.make_async_copy(*acc_desc(k)).wait()

                start_idx(g + LAG, (k + LAG) % RB)

                wait_idx(k)
                compute_sel(k)
                pltpu.async_copy(*gather_desc(k))

                kp = (k - LAG) % RB
                if k >= LAG:
                    pltpu.make_async_copy(*gather_desc(kp)).wait()
                    pltpu.async_copy(*acc_desc(kp), add=True)
                else:
                    @pl.when(G >= 1)
                    def _():
                        pltpu.make_async_copy(*gather_desc(kp)).wait()
                        pltpu.async_copy(*acc_desc(kp), add=True)

        for k in range(LAG):
            pltpu.make_async_copy(*acc_desc(k)).wait()
            wait_idx(k)
            compute_sel(k)
            pltpu.async_copy(*gather_desc(k))
        for k in range(LAG, RB):
            pltpu.make_async_copy(*gather_desc(k)).wait()
            pltpu.async_copy(*acc_desc(k), add=True)
        for k in range(LAG):
            pltpu.make_async_copy(*gather_desc(k)).wait()
            pltpu.async_copy(*acc_desc(k), add=True)
        for k in range(RB):
            pltpu.make_async_copy(*acc_desc(k)).wait()

        @pl.loop(0, CHUNK)
        def _(r):
            for j in range(D // 16):
                rows[0][r, pl.ds(j * 16, 16)] = ones16

        for g in range(LAG):
            start_idx(g, g, col_only=True)

        @pl.loop(0, N_SUPER)
        def _(G):
            for k in range(RB):
                g = G * RB + k

                @pl.when(G >= 1)
                def _():
                    pltpu.make_async_copy(*cnt_desc(k)).wait()

                start_idx(g + LAG, (k + LAG) % RB, col_only=True)

                wait_idx(k, col_only=True)
                compute_sel(k, col_only=True)
                pltpu.async_copy(*cnt_desc(k), add=True)

        for k in range(LAG):
            pltpu.make_async_copy(*cnt_desc(k)).wait()
            wait_idx(k, col_only=True)
            compute_sel(k, col_only=True)
            pltpu.async_copy(*cnt_desc(k), add=True)
        for k in range(RB):
            pltpu.make_async_copy(*cnt_desc(k)).wait()

        plsc.subcore_barrier()

        out_rows = pl.ds(row_base, ROWS_PER_TILE)
        pltpu.sync_copy(acc_s.at[out_rows], acc_out.at[cid, out_rows])
        pltpu.sync_copy(cnt_s.at[out_rows], cnt_out.at[cid, out_rows])

    return sc_kernel(x_src, row_idx, col_idx)


BN = 2000


def _tc_combine_kernel(acc_ref, cnt_ref, xd_ref, wl_ref, bl_ref, wr_ref, o_ref):
    cnt = cnt_ref[:, 0:1]
    aggr = acc_ref[...] / jnp.maximum(cnt, 1.0)
    dn = (((1,), (1,)), ((), ()))
    out = lax.dot_general(aggr, wl_ref[...], dn, preferred_element_type=jnp.float32)
    out += lax.dot_general(xd_ref[...], wr_ref[...], dn, preferred_element_type=jnp.float32)
    o_ref[...] = out + bl_ref[...]


def _tc_combine(acc, cnt, x_dst, W_l, b_l, W_r):
    return pl.pallas_call(
        _tc_combine_kernel,
        grid=(N_NODES // BN,),
        in_specs=[
            pl.BlockSpec((BN, D), lambda i: (i, 0)),
            pl.BlockSpec((BN, D), lambda i: (i, 0)),
            pl.BlockSpec((BN, D), lambda i: (i, 0)),
            pl.BlockSpec((D, D), lambda i: (0, 0)),
            pl.BlockSpec((1, D), lambda i: (0, 0)),
            pl.BlockSpec((D, D), lambda i: (0, 0)),
        ],
        out_specs=pl.BlockSpec((BN, D), lambda i: (i, 0)),
        out_shape=jax.ShapeDtypeStruct((N_NODES, D), jnp.float32),
    )(acc, cnt, x_dst, W_l, b_l, W_r)


def kernel(x_src, x_dst, W_l, b_l, W_r, edge_index):
    row_idx = edge_index[0].astype(jnp.int32)
    col_idx = edge_index[1].astype(jnp.int32)
    acc, cnt = _sc_aggregate(x_src, row_idx, col_idx)
    acc = acc.reshape(N_PAD, D)
    cnt = cnt.reshape(N_PAD, D)
    return _tc_combine(acc, cnt, x_dst, W_l, b_l.reshape(1, D), W_r)


# speedup vs baseline: 9.7291x; 1.0924x over previous
import functools

import jax
import jax.numpy as jnp
from jax import lax
from jax.experimental import pallas as pl
from jax.experimental.pallas import tpu as pltpu
from jax.experimental.pallas import tpu_sc as plsc

N_NODES = 10000
N_EDGES = 320000
D = 128

NC = 2
NS = 16
E_PER_T = N_EDGES // NS
CHUNK = 80
N_CHUNKS = E_PER_T // CHUNK
RB = 4
N_SUPER = (N_CHUNKS - 2) // RB
LAG = 2
N_PAD = 10240
HALF = N_PAD // NC
ROWS_PER_TILE = HALF // NS
ZROWS = 32
IGN = 2**31 - 1


def _sc_aggregate(x_src, row_idx, col_idx):
    mesh = plsc.VectorSubcoreMesh(core_axis_name="c", subcore_axis_name="s")

    @functools.partial(
        pl.kernel,
        out_type=(
            jax.ShapeDtypeStruct((NC, HALF, D), jnp.float32),
            jax.ShapeDtypeStruct((NC, HALF, D), jnp.float32),
        ),
        mesh=mesh,
        scratch_types=[
            [pltpu.VMEM((CHUNK,), jnp.int32)] * RB,
            [pltpu.VMEM((CHUNK,), jnp.int32)] * RB,
            [pltpu.VMEM((CHUNK,), jnp.int32)] * RB,
            [pltpu.VMEM((CHUNK,), jnp.int32)] * RB,
            [pltpu.VMEM((32,), jnp.int32)] * RB,
            [pltpu.VMEM((32,), jnp.int32)] * RB,
            [pltpu.VMEM((16,), jnp.int32)] * RB,
            [pltpu.VMEM((CHUNK, D), jnp.float32)] * RB,
            pltpu.VMEM((ZROWS, D), jnp.float32),
            pltpu.VMEM_SHARED((HALF, D), jnp.float32),
            pltpu.VMEM_SHARED((HALF, D), jnp.float32),
            [pltpu.SemaphoreType.DMA] * RB,
            [pltpu.SemaphoreType.DMA] * RB,
            [pltpu.SemaphoreType.DMA] * RB,
            [pltpu.SemaphoreType.DMA] * RB,
        ],
    )
    def sc_kernel(x_hbm, row_hbm, col_hbm, acc_out, cnt_out,
                  rowi, coli, selr, selc, selca, selcb, selcc, rows, zbuf_v,
                  acc_s, cnt_s, isem, gsem, asem, csem):
        cid = lax.axis_index("c")
        sid = lax.axis_index("s")
        zeros16 = jnp.zeros((16,), jnp.float32)
        ones16 = jnp.ones((16,), jnp.float32)
        row_base = sid * ROWS_PER_TILE
        lo = cid * HALF
        e_base = sid * E_PER_T

        def start_idx(g, k, col_only=False):
            sl = pl.ds(e_base + g * CHUNK, CHUNK)
            pltpu.async_copy(col_hbm.at[sl], coli[k], isem[k])
            if not col_only:
                pltpu.async_copy(row_hbm.at[sl], rowi[k], isem[k])

        def wait_idx(k, col_only=False):
            pltpu.make_async_copy(col_hbm.at[pl.ds(0, CHUNK)], coli[k],
                                  isem[k]).wait()
            if not col_only:
                pltpu.make_async_copy(row_hbm.at[pl.ds(0, CHUNK)], rowi[k],
                                      isem[k]).wait()

        def compute_sel(k):
            for j in range(CHUNK // 16):
                lanes = pl.ds(j * 16, 16)
                c16 = coli[k][lanes]
                owned = (c16 >= lo) & (c16 < lo + HALF)
                sel = jnp.where(owned, c16 - lo, IGN)
                selc[k][lanes] = sel
                if j < 2:
                    selca[k][lanes] = sel
                elif j < 4:
                    selcb[k][pl.ds((j - 2) * 16, 16)] = sel
                else:
                    selcc[k][pl.ds(0, 16)] = sel
                selr[k][lanes] = jnp.where(owned, rowi[k][lanes], IGN)

        def gather_desc(k):
            return (x_hbm.at[plsc.Indices(selr[k], ignored_value=IGN)],
                    rows[k], gsem[k])

        def acc_desc(k):
            return (rows[k], acc_s.at[plsc.Indices(selc[k], ignored_value=IGN)],
                    asem[k])

        def cnt_descs(k):
            return (
                (zbuf_v, cnt_s.at[plsc.Indices(selca[k], ignored_value=IGN)],
                 csem[k]),
                (zbuf_v, cnt_s.at[plsc.Indices(selcb[k], ignored_value=IGN)],
                 csem[k]),
                (zbuf_v.at[pl.ds(0, 16)],
                 cnt_s.at[plsc.Indices(selcc[k], ignored_value=IGN)],
                 csem[k]),
            )

        def start_cnt(k):
            for d in cnt_descs(k):
                pltpu.async_copy(*d, add=True)

        def wait_cnt(k):
            for d in cnt_descs(k):
                pltpu.make_async_copy(*d).wait()

        @pl.loop(0, ZROWS)
        def _(r):
            for j in range(D // 16):
                zbuf_v[r, pl.ds(j * 16, 16)] = zeros16

        @pl.loop(0, ROWS_PER_TILE // ZROWS)
        def _(i):
            pltpu.sync_copy(zbuf_v, acc_s.at[pl.ds(row_base + i * ZROWS, ZROWS)])
            pltpu.sync_copy(zbuf_v, cnt_s.at[pl.ds(row_base + i * ZROWS, ZROWS)])

        @pl.loop(0, ZROWS)
        def _(r):
            for j in range(D // 16):
                zbuf_v[r, pl.ds(j * 16, 16)] = ones16

        plsc.subcore_barrier()

        for g in range(LAG):
            start_idx(g, g)

        @pl.loop(0, N_SUPER)
        def _(G):
            for k in range(RB):
                g = G * RB + k

                @pl.when(G >= 1)
                def _():
                    pltpu.make_async_copy(*acc_desc(k)).wait()
                    wait_cnt(k)

                start_idx(g + LAG, (k + LAG) % RB)

                wait_idx(k)
                compute_sel(k)
                pltpu.async_copy(*gather_desc(k))

                kp = (k - LAG) % RB
                if k >= LAG:
                    pltpu.make_async_copy(*gather_desc(kp)).wait()
                    pltpu.async_copy(*acc_desc(kp), add=True)
                    start_cnt(kp)
                else:
                    @pl.when(G >= 1)
                    def _():
                        pltpu.make_async_copy(*gather_desc(kp)).wait()
                        pltpu.async_copy(*acc_desc(kp), add=True)
                        start_cnt(kp)

        for k in range(LAG):
            pltpu.make_async_copy(*acc_desc(k)).wait()
            wait_cnt(k)
            wait_idx(k)
            compute_sel(k)
            pltpu.async_copy(*gather_desc(k))
        for k in range(LAG, RB):
            pltpu.make_async_copy(*gather_desc(k)).wait()
            pltpu.async_copy(*acc_desc(k), add=True)
            start_cnt(k)
        for k in range(LAG):
            pltpu.make_async_copy(*gather_desc(k)).wait()
            pltpu.async_copy(*acc_desc(k), add=True)
            start_cnt(k)
        for k in range(RB):
            pltpu.make_async_copy(*acc_desc(k)).wait()
            wait_cnt(k)

        plsc.subcore_barrier()

        out_rows = pl.ds(row_base, ROWS_PER_TILE)
        pltpu.sync_copy(acc_s.at[out_rows], acc_out.at[cid, out_rows])
        pltpu.sync_copy(cnt_s.at[out_rows], cnt_out.at[cid, out_rows])

    return sc_kernel(x_src, row_idx, col_idx)


BN = 2000


def _tc_combine_kernel(acc_ref, cnt_ref, xd_ref, wl_ref, bl_ref, wr_ref, o_ref):
    cnt = cnt_ref[:, 0:1]
    aggr = acc_ref[...] / jnp.maximum(cnt, 1.0)
    dn = (((1,), (1,)), ((), ()))
    out = lax.dot_general(aggr, wl_ref[...], dn, preferred_element_type=jnp.float32)
    out += lax.dot_general(xd_ref[...], wr_ref[...], dn, preferred_element_type=jnp.float32)
    o_ref[...] = out + bl_ref[...]


def _tc_combine(acc, cnt, x_dst, W_l, b_l, W_r):
    return pl.pallas_call(
        _tc_combine_kernel,
        grid=(N_NODES // BN,),
        in_specs=[
            pl.BlockSpec((BN, D), lambda i: (i, 0)),
            pl.BlockSpec((BN, D), lambda i: (i, 0)),
            pl.BlockSpec((BN, D), lambda i: (i, 0)),
            pl.BlockSpec((D, D), lambda i: (0, 0)),
            pl.BlockSpec((1, D), lambda i: (0, 0)),
            pl.BlockSpec((D, D), lambda i: (0, 0)),
        ],
        out_specs=pl.BlockSpec((BN, D), lambda i: (i, 0)),
        out_shape=jax.ShapeDtypeStruct((N_NODES, D), jnp.float32),
    )(acc, cnt, x_dst, W_l, b_l, W_r)


def kernel(x_src, x_dst, W_l, b_l, W_r, edge_index):
    row_idx = edge_index[0].astype(jnp.int32)
    col_idx = edge_index[1].astype(jnp.int32)
    acc, cnt = _sc_aggregate(x_src, row_idx, col_idx)
    acc = acc.reshape(N_PAD, D)
    cnt = cnt.reshape(N_PAD, D)
    return _tc_combine(acc, cnt, x_dst, W_l, b_l.reshape(1, D), W_r)
